# baseline (speedup 1.0000x reference)
import functools

import jax
import jax.numpy as jnp
from jax import lax
from jax.experimental import pallas as pl
from jax.experimental.pallas import tpu as pltpu
from jax.experimental.pallas import tpu_sc as plsc

N = 10000
E = 320000
D_IN = 128
D_HID = 128
D_OUT = 64

NC = 2
NS = 16
B = 128
EROWS = E // B
ROWS_PER_TILE = 79
LAST_ROWS = EROWS - (NC * NS - 1) * ROWS_PER_TILE

NPAD = 10240
NPT = NPAD // NS

BN = 512
GRID = (N + BN - 1) // BN

DEG_ROWS = EROWS // NS
DEG_EXTRA = EROWS % NS


def _mesh():
  return plsc.VectorSubcoreMesh(core_axis_name="c", subcore_axis_name="s")


def _fill(ref, n16, value):
  def body(i, _):
    ref[pl.ds(i * 16, 16)] = jnp.full((16,), value, jnp.float32)
    return 0

  lax.fori_loop(0, n16, body, 0)


def _bcast16(v, k):
  dnums = lax.GatherDimensionNumbers(
      offset_dims=(), collapsed_slice_dims=(0,), start_index_map=(0,))
  return lax.gather(v, jnp.full((16, 1), k, jnp.int32), dnums, (1,),
                    mode=lax.GatherScatterMode.PROMISE_IN_BOUNDS)


def _newton_rsqrt16(x):
  s = 0.5 * (x + 1.0)
  for _ in range(20):
    s = 0.5 * (s + x / s)
  return 1.0 / s


def _agg_loop(ei_hbm, gsrc, acc_s, stg, isl, gsem, ssem, isem, row0, nb):
  pltpu.sync_copy(ei_hbm.at[:, row0], isl[0])
  pltpu.async_copy(ei_hbm.at[:, row0 + 1], isl[1], isem[1])
  pltpu.async_copy(gsrc.at[isl[0].at[0]], stg[0], gsem[0])
  plsc.subcore_barrier()

  def group(gidx, _):
    j0 = gidx * 6
    for b in range(6):
      j = j0 + b
      sb = b % 2
      il = b % 3

      @pl.when(j < nb)
      def _():
        @pl.when(j >= 1)
        def _():
          pltpu.make_async_copy(stg[1 - sb], acc_s.at[isl[il].at[1]],
                                ssem[1 - sb]).wait()

        @pl.when(j + 1 < nb)
        def _():
          pltpu.make_async_copy(ei_hbm.at[:, row0], isl[(il + 1) % 3],
                                isem[(il + 1) % 3]).wait()
          pltpu.async_copy(gsrc.at[isl[(il + 1) % 3].at[0]], stg[1 - sb],
                           gsem[1 - sb])

        @pl.when(j + 2 < nb)
        def _():
          pltpu.async_copy(ei_hbm.at[:, row0 + j + 2], isl[(il + 2) % 3],
                           isem[(il + 2) % 3])

        pltpu.make_async_copy(gsrc.at[isl[il].at[0]], stg[sb],
                              gsem[sb]).wait()
        pltpu.async_copy(stg[sb], acc_s.at[isl[il].at[1]], ssem[sb],
                         add=True)

    return 0

  lax.fori_loop(0, (ROWS_PER_TILE + 5) // 6, group, 0)
  pltpu.make_async_copy(stg[0], acc_s.at[isl[0].at[1]], ssem[0]).wait()


DEG_MAX = DEG_ROWS + 1


def _deg_body(ei_hbm, dinv_hbm,
              idx_all, ones_v, zb, degv, dinvv, deg_s, dsem):
  cid = lax.axis_index("c")
  sid = lax.axis_index("s")
  base = sid * NPT

  _fill(ones_v, B // 16, 1.0)
  _fill(zb, NPT // 16, 0.0)
  pltpu.sync_copy(zb, deg_s.at[pl.ds(base, NPT)])

  nbd = jnp.where(sid < DEG_EXTRA, DEG_ROWS + 1, DEG_ROWS)
  row0d = sid * DEG_ROWS + jnp.minimum(sid, DEG_EXTRA)
  row0c = jnp.minimum(row0d, EROWS - DEG_MAX)
  off = row0d - row0c
  pltpu.sync_copy(ei_hbm.at[:, pl.ds(row0c, DEG_MAX)], idx_all)
  plsc.subcore_barrier()

  def fire(j, _):
    pltpu.async_copy(ones_v, deg_s.at[idx_all.at[1, off + j]], dsem,
                     add=True)
    return 0

  lax.fori_loop(0, nbd, fire, 0)

  def drain(j, _):
    pltpu.make_async_copy(ones_v, deg_s.at[idx_all.at[1, 0]], dsem).wait()
    return 0

  lax.fori_loop(0, nbd, drain, 0)
  plsc.subcore_barrier()

  pltpu.sync_copy(deg_s.at[pl.ds(base, NPT)], degv)

  def newton(i, _):
    d = degv[pl.ds(i * 16, 16)] + 1.0
    dinvv[pl.ds(i * 16, 16)] = _newton_rsqrt16(d)
    return 0

  lax.fori_loop(0, NPT // 16, newton, 0)

  @pl.when(cid == 0)
  def _():
    pltpu.sync_copy(dinvv, dinv_hbm.at[pl.ds(base, NPT)])


def _sc_degree(ei3):
  k = pl.kernel(
      _deg_body,
      out_type=jax.ShapeDtypeStruct((NPAD,), jnp.float32),
      mesh=_mesh(),
      compiler_params=pltpu.CompilerParams(use_tc_tiling_on_sc=False),
      scratch_types=(
          [pltpu.VMEM((2, DEG_MAX, B), jnp.int32)]
          + [pltpu.VMEM((B,), jnp.float32)]
          + [pltpu.VMEM((NPT,), jnp.float32)]
          + [pltpu.VMEM((NPT,), jnp.float32)]
          + [pltpu.VMEM((NPT,), jnp.float32)]
          + [pltpu.VMEM_SHARED((NPAD,), jnp.float32)]
          + [pltpu.SemaphoreType.DMA]
      ),
  )
  return k(ei3)


def _agg_body(g_hbm, ei_hbm, out_hbm,
              stg0, stg1, isl0, isl1, isl2, acc_s, g_s,
              gs0, gs1, ss0, ss1, is0, is1, is2, D, spmem_src):
  stg = (stg0, stg1)
  isl = (isl0, isl1, isl2)
  cid = lax.axis_index("c")
  sid = lax.axis_index("s")
  wid = cid * NS + sid

  def fill_stg(i, _):
    r = i // (D // 16)
    c = i % (D // 16)
    stg0[r, pl.ds(c * 16, 16)] = jnp.zeros((16,), jnp.float32)
    return 0

  lax.fori_loop(0, B * (D // 16), fill_stg, 0)
  base = sid * NPT
  for kk in range(NPT // B):
    pltpu.sync_copy(stg0, acc_s.at[pl.ds(base + kk * B, B)])

  if spmem_src:
    pltpu.sync_copy(g_hbm.at[pl.ds(sid * 625, 625)],
                    g_s.at[pl.ds(sid * 625, 625)])
    gsrc = g_s
  else:
    gsrc = g_hbm
  plsc.subcore_barrier()

  nb = jnp.where(wid == NC * NS - 1, LAST_ROWS, ROWS_PER_TILE)
  row0 = wid * ROWS_PER_TILE
  _agg_loop(ei_hbm, gsrc, acc_s, stg, isl, (gs0, gs1), (ss0, ss1),
            (is0, is1, is2), row0, nb)

  plsc.subcore_barrier()
  pltpu.sync_copy(acc_s.at[pl.ds(base, NPT)],
                  out_hbm.at[cid, pl.ds(base, NPT)])


def _sc_aggregate(g, ei3, D, spmem_src=False):
  k = pl.kernel(
      functools.partial(_agg_body, D=D, spmem_src=spmem_src),
      out_type=jax.ShapeDtypeStruct((NC, NPAD, D), jnp.float32),
      mesh=_mesh(),
      compiler_params=pltpu.CompilerParams(use_tc_tiling_on_sc=False),
      scratch_types=(
          [pltpu.VMEM((B, D), jnp.float32) for _ in range(2)]
          + [pltpu.VMEM((2, B), jnp.int32) for _ in range(3)]
          + [pltpu.VMEM_SHARED((NPAD, D), jnp.float32)]
          + [pltpu.VMEM_SHARED((N if spmem_src else 8, D), jnp.float32)]
          + [pltpu.SemaphoreType.DMA for _ in range(7)]
      ),
  )
  return k(g, ei3)


def _tc1_body(dinv_ref, x_ref, w1_ref, g1_ref):
  h = jnp.dot(x_ref[...], w1_ref[...], preferred_element_type=jnp.float32)
  g1_ref[...] = h * dinv_ref[...]


def _tc1(dinv, x, W1):
  return pl.pallas_call(
      _tc1_body,
      grid=(GRID,),
      in_specs=[
          pl.BlockSpec((BN, 1), lambda i: (i, 0)),
          pl.BlockSpec((BN, D_IN), lambda i: (i, 0)),
          pl.BlockSpec((D_IN, D_HID), lambda i: (0, 0)),
      ],
      out_specs=pl.BlockSpec((BN, D_HID), lambda i: (i, 0)),
      out_shape=jax.ShapeDtypeStruct((N, D_HID), jnp.float32),
  )(dinv, x, W1)


def _tc2_body(dinv_ref, accp_ref, g1_ref, w2_ref, b1_ref, g2_ref):
  dinv = dinv_ref[...]
  agg = accp_ref[0] + accp_ref[1] + g1_ref[...]
  t = jnp.maximum(agg * dinv + b1_ref[...], 0.0)
  h2 = jnp.dot(t, w2_ref[...], preferred_element_type=jnp.float32)
  g2_ref[...] = h2 * dinv


def _tc2(dinv, accp, g1, W2, b1):
  return pl.pallas_call(
      _tc2_body,
      grid=(GRID,),
      in_specs=[
          pl.BlockSpec((BN, 1), lambda i: (i, 0)),
          pl.BlockSpec((NC, BN, D_HID), lambda i: (0, i, 0)),
          pl.BlockSpec((BN, D_HID), lambda i: (i, 0)),
          pl.BlockSpec((D_HID, D_OUT), lambda i: (0, 0)),
          pl.BlockSpec((D_HID,), lambda i: (0,)),
      ],
      out_specs=pl.BlockSpec((BN, D_OUT), lambda i: (i, 0)),
      out_shape=jax.ShapeDtypeStruct((N, D_OUT), jnp.float32),
  )(dinv, accp, g1, W2, b1)


def _tc3_body(dinv_ref, accp_ref, g2_ref, b2_ref, fcw_ref, fcb_ref, out_ref):
  dinv = dinv_ref[...]
  agg = accp_ref[0] + accp_ref[1] + g2_ref[...]
  t = agg * dinv + b2_ref[...]
  o = jnp.dot(t, fcw_ref[...], preferred_element_type=jnp.float32)
  out_ref[...] = 1.0 / (1.0 + jnp.exp(-(o + fcb_ref[...])))


def _tc3(dinv, accp, g2, b2, fc_W, fc_b):
  return pl.pallas_call(
      _tc3_body,
      grid=(GRID,),
      in_specs=[
          pl.BlockSpec((BN, 1), lambda i: (i, 0)),
          pl.BlockSpec((NC, BN, D_OUT), lambda i: (0, i, 0)),
          pl.BlockSpec((BN, D_OUT), lambda i: (i, 0)),
          pl.BlockSpec((D_OUT,), lambda i: (0,)),
          pl.BlockSpec((D_OUT, 1), lambda i: (0, 0)),
          pl.BlockSpec((1,), lambda i: (0,)),
      ],
      out_specs=pl.BlockSpec((BN, 1), lambda i: (i, 0)),
      out_shape=jax.ShapeDtypeStruct((N, 1), jnp.float32),
  )(dinv, accp, g2, b2, fc_W, fc_b)


def kernel(x, edge_index, W1, b1, W2, b2, fc_W, fc_b):
  ei3 = edge_index.astype(jnp.int32).reshape(2, EROWS, B)

  dinvf = _sc_degree(ei3)
  dinv = dinvf[:N].reshape(N, 1)
  g1 = _tc1(dinv, x, W1)
  acc1 = _sc_aggregate(g1, ei3, D_HID)
  g2 = _tc2(dinv, acc1, g1, W2, b1)
  acc2 = _sc_aggregate(g2, ei3, D_OUT, spmem_src=True)
  return _tc3(dinv, acc2, g2, b2, fc_W, fc_b)


# speedup vs baseline: 1.0529x; 1.0529x over previous
import functools

import jax
import jax.numpy as jnp
from jax import lax
from jax.experimental import pallas as pl
from jax.experimental.pallas import tpu as pltpu
from jax.experimental.pallas import tpu_sc as plsc

N = 10000
E = 320000
D_IN = 128
D_HID = 128
D_OUT = 64

NC = 2
NS = 16
B = 128
EROWS = E // B
ROWS_PER_TILE = 79
LAST_ROWS = EROWS - (NC * NS - 1) * ROWS_PER_TILE

NPAD = 10240
NPT = NPAD // NS

BN = 512
GRID = (N + BN - 1) // BN

DEG_ROWS = EROWS // NS
DEG_EXTRA = EROWS % NS


def _mesh():
  return plsc.VectorSubcoreMesh(core_axis_name="c", subcore_axis_name="s")


def _fill(ref, n16, value):
  def body(i, _):
    ref[pl.ds(i * 16, 16)] = jnp.full((16,), value, jnp.float32)
    return 0

  lax.fori_loop(0, n16, body, 0)


def _bcast16(v, k):
  dnums = lax.GatherDimensionNumbers(
      offset_dims=(), collapsed_slice_dims=(0,), start_index_map=(0,))
  return lax.gather(v, jnp.full((16, 1), k, jnp.int32), dnums, (1,),
                    mode=lax.GatherScatterMode.PROMISE_IN_BOUNDS)


def _newton_rsqrt16(x):
  s = 0.5 * (x + 1.0)
  for _ in range(20):
    s = 0.5 * (s + x / s)
  return 1.0 / s


def _agg_loop(ei_hbm, gsrc, acc_s, stg, isl, gsem, ssem, isem, row0, nb):
  pltpu.sync_copy(ei_hbm.at[:, row0], isl[0])
  pltpu.async_copy(ei_hbm.at[:, row0 + 1], isl[1], isem[1])
  pltpu.async_copy(gsrc.at[isl[0].at[0]], stg[0], gsem[0])
  plsc.subcore_barrier()

  def group(gidx, _):
    j0 = gidx * 6
    for b in range(6):
      j = j0 + b
      sb = b % 2
      il = b % 3

      @pl.when(j < nb)
      def _():
        @pl.when(j >= 1)
        def _():
          pltpu.make_async_copy(stg[1 - sb], acc_s.at[isl[il].at[1]],
                                ssem[1 - sb]).wait()

        @pl.when(j + 1 < nb)
        def _():
          pltpu.make_async_copy(ei_hbm.at[:, row0], isl[(il + 1) % 3],
                                isem[(il + 1) % 3]).wait()
          pltpu.async_copy(gsrc.at[isl[(il + 1) % 3].at[0]], stg[1 - sb],
                           gsem[1 - sb])

        @pl.when(j + 2 < nb)
        def _():
          pltpu.async_copy(ei_hbm.at[:, row0 + j + 2], isl[(il + 2) % 3],
                           isem[(il + 2) % 3])

        pltpu.make_async_copy(gsrc.at[isl[il].at[0]], stg[sb],
                              gsem[sb]).wait()
        pltpu.async_copy(stg[sb], acc_s.at[isl[il].at[1]], ssem[sb],
                         add=True)

    return 0

  lax.fori_loop(0, (ROWS_PER_TILE + 5) // 6, group, 0)
  pltpu.make_async_copy(stg[0], acc_s.at[isl[0].at[1]], ssem[0]).wait()


DEG_MAX = DEG_ROWS + 1


def _deg_body(ei_hbm, dinv_hbm,
              idx_all, ones_v, zb, degv, dinvv, deg_s, dsem):
  cid = lax.axis_index("c")
  sid = lax.axis_index("s")
  base = sid * NPT

  _fill(ones_v, B // 16, 1.0)
  _fill(zb, NPT // 16, 0.0)
  pltpu.sync_copy(zb, deg_s.at[pl.ds(base, NPT)])

  nbd = jnp.where(sid < DEG_EXTRA, DEG_ROWS + 1, DEG_ROWS)
  row0d = sid * DEG_ROWS + jnp.minimum(sid, DEG_EXTRA)
  row0c = jnp.minimum(row0d, EROWS - DEG_MAX)
  off = row0d - row0c
  pltpu.sync_copy(ei_hbm.at[:, pl.ds(row0c, DEG_MAX)], idx_all)
  plsc.subcore_barrier()

  def fire(j, _):
    pltpu.async_copy(ones_v, deg_s.at[idx_all.at[1, off + j]], dsem,
                     add=True)
    return 0

  lax.fori_loop(0, nbd, fire, 0)

  def drain(j, _):
    pltpu.make_async_copy(ones_v, deg_s.at[idx_all.at[1, 0]], dsem).wait()
    return 0

  lax.fori_loop(0, nbd, drain, 0)
  plsc.subcore_barrier()

  pltpu.sync_copy(deg_s.at[pl.ds(base, NPT)], degv)

  def newton(i, _):
    d = degv[pl.ds(i * 16, 16)] + 1.0
    dinvv[pl.ds(i * 16, 16)] = _newton_rsqrt16(d)
    return 0

  lax.fori_loop(0, NPT // 16, newton, 0)

  @pl.when(cid == 0)
  def _():
    pltpu.sync_copy(dinvv, dinv_hbm.at[pl.ds(base, NPT)])


def _sc_degree(ei3):
  k = pl.kernel(
      _deg_body,
      out_type=jax.ShapeDtypeStruct((NPAD,), jnp.float32),
      mesh=_mesh(),
      compiler_params=pltpu.CompilerParams(use_tc_tiling_on_sc=False),
      scratch_types=(
          [pltpu.VMEM((2, DEG_MAX, B), jnp.int32)]
          + [pltpu.VMEM((B,), jnp.float32)]
          + [pltpu.VMEM((NPT,), jnp.float32)]
          + [pltpu.VMEM((NPT,), jnp.float32)]
          + [pltpu.VMEM((NPT,), jnp.float32)]
          + [pltpu.VMEM_SHARED((NPAD,), jnp.float32)]
          + [pltpu.SemaphoreType.DMA]
      ),
  )
  return k(ei3)


def _agg_body(g_hbm, ei_hbm, out_hbm,
              stg0, stg1, isl0, isl1, isl2, acc_s,
              gs0, gs1, ss0, ss1, is0, is1, is2, D):
  stg = (stg0, stg1)
  isl = (isl0, isl1, isl2)
  cid = lax.axis_index("c")
  sid = lax.axis_index("s")
  wid = cid * NS + sid

  def fill_stg(i, _):
    r = i // (D // 16)
    c = i % (D // 16)
    stg0[r, pl.ds(c * 16, 16)] = jnp.zeros((16,), jnp.float32)
    return 0

  lax.fori_loop(0, B * (D // 16), fill_stg, 0)
  base = sid * NPT
  for kk in range(NPT // B):
    pltpu.sync_copy(stg0, acc_s.at[pl.ds(base + kk * B, B)])

  nb = jnp.where(wid == NC * NS - 1, LAST_ROWS, ROWS_PER_TILE)
  row0 = wid * ROWS_PER_TILE
  _agg_loop(ei_hbm, g_hbm, acc_s, stg, isl, (gs0, gs1), (ss0, ss1),
            (is0, is1, is2), row0, nb)

  plsc.subcore_barrier()
  pltpu.sync_copy(acc_s.at[pl.ds(base, NPT)],
                  out_hbm.at[cid, pl.ds(base, NPT)])


def _sc_aggregate(g, ei3, D):
  k = pl.kernel(
      functools.partial(_agg_body, D=D),
      out_type=jax.ShapeDtypeStruct((NC, NPAD, D), jnp.float32),
      mesh=_mesh(),
      compiler_params=pltpu.CompilerParams(use_tc_tiling_on_sc=False),
      scratch_types=(
          [pltpu.VMEM((B, D), jnp.float32) for _ in range(2)]
          + [pltpu.VMEM((2, B), jnp.int32) for _ in range(3)]
          + [pltpu.VMEM_SHARED((NPAD, D), jnp.float32)]
          + [pltpu.SemaphoreType.DMA for _ in range(7)]
      ),
  )
  return k(g, ei3)


def _tc1_body(dinv_ref, x_ref, w1_ref, g1_ref):
  h = jnp.dot(x_ref[...], w1_ref[...], preferred_element_type=jnp.float32)
  g1_ref[...] = h * dinv_ref[...]


def _tc1(dinv, x, W1):
  return pl.pallas_call(
      _tc1_body,
      grid=(GRID,),
      in_specs=[
          pl.BlockSpec((BN, 1), lambda i: (i, 0)),
          pl.BlockSpec((BN, D_IN), lambda i: (i, 0)),
          pl.BlockSpec((D_IN, D_HID), lambda i: (0, 0)),
      ],
      out_specs=pl.BlockSpec((BN, D_HID), lambda i: (i, 0)),
      out_shape=jax.ShapeDtypeStruct((N, D_HID), jnp.float32),
  )(dinv, x, W1)


def _tc2_body(dinv_ref, accp_ref, g1_ref, w2_ref, b1_ref, g2_ref):
  dinv = dinv_ref[...]
  agg = accp_ref[0] + accp_ref[1] + g1_ref[...]
  t = jnp.maximum(agg * dinv + b1_ref[...], 0.0)
  h2 = jnp.dot(t, w2_ref[...], preferred_element_type=jnp.float32)
  g2_ref[...] = h2 * dinv


def _tc2(dinv, accp, g1, W2, b1):
  return pl.pallas_call(
      _tc2_body,
      grid=(GRID,),
      in_specs=[
          pl.BlockSpec((BN, 1), lambda i: (i, 0)),
          pl.BlockSpec((NC, BN, D_HID), lambda i: (0, i, 0)),
          pl.BlockSpec((BN, D_HID), lambda i: (i, 0)),
          pl.BlockSpec((D_HID, D_OUT), lambda i: (0, 0)),
          pl.BlockSpec((D_HID,), lambda i: (0,)),
      ],
      out_specs=pl.BlockSpec((BN, D_OUT), lambda i: (i, 0)),
      out_shape=jax.ShapeDtypeStruct((N, D_OUT), jnp.float32),
  )(dinv, accp, g1, W2, b1)


def _tc3_body(dinv_ref, accp_ref, g2_ref, b2_ref, fcw_ref, fcb_ref, out_ref):
  dinv = dinv_ref[...]
  agg = accp_ref[0] + accp_ref[1] + g2_ref[...]
  t = agg * dinv + b2_ref[...]
  o = jnp.dot(t, fcw_ref[...], preferred_element_type=jnp.float32)
  out_ref[...] = 1.0 / (1.0 + jnp.exp(-(o + fcb_ref[...])))


def _tc3(dinv, accp, g2, b2, fc_W, fc_b):
  return pl.pallas_call(
      _tc3_body,
      grid=(GRID,),
      in_specs=[
          pl.BlockSpec((BN, 1), lambda i: (i, 0)),
          pl.BlockSpec((NC, BN, D_OUT), lambda i: (0, i, 0)),
          pl.BlockSpec((BN, D_OUT), lambda i: (i, 0)),
          pl.BlockSpec((D_OUT,), lambda i: (0,)),
          pl.BlockSpec((D_OUT, 1), lambda i: (0, 0)),
          pl.BlockSpec((1,), lambda i: (0,)),
      ],
      out_specs=pl.BlockSpec((BN, 1), lambda i: (i, 0)),
      out_shape=jax.ShapeDtypeStruct((N, 1), jnp.float32),
  )(dinv, accp, g2, b2, fc_W, fc_b)


def kernel(x, edge_index, W1, b1, W2, b2, fc_W, fc_b):
  ei3 = edge_index.astype(jnp.int32).reshape(2, EROWS, B)

  dinvf = _sc_degree(ei3)
  dinv = dinvf[:N].reshape(N, 1)
  g1 = _tc1(dinv, x, W1)
  acc1 = _sc_aggregate(g1, ei3, D_HID)
  g2 = _tc2(dinv, acc1, g1, W2, b1)
  acc2 = _sc_aggregate(g2, ei3, D_OUT)
  return _tc3(dinv, acc2, g2, b2, fc_W, fc_b)
